# baseline (speedup 1.0000x reference)
import functools

import jax
import jax.numpy as jnp
from jax import lax
from jax.experimental import pallas as pl
from jax.experimental.pallas import tpu as pltpu
from jax.experimental.pallas import tpu_sc as plsc

N = 10000
NP = 10240
D = 128
DO = 64
NC, NS, L = 2, 16, 16
NW = NC * NS
NB = 81
EPW = NB * 128
E_PAD = EPW * NW
E_AUG = 320000 + N
RPT = NP // NS
AP_STRIDE = 82944
AP = NS * AP_STRIDE

_mesh = plsc.VectorSubcoreMesh(core_axis_name="c", subcore_axis_name="s")


def _fill(ref, n_vregs, value, dtype):
    def body(i, _):
        ref[pl.ds(i * L, L)] = jnp.full((L,), value, dtype)
        return 0
    lax.fori_loop(0, n_vregs, body, 0)


@functools.partial(
    pl.kernel,
    out_type=jax.ShapeDtypeStruct((NC, NP), jnp.float32),
    mesh=_mesh,
    scratch_types=[
        pltpu.VMEM((NB, 128), jnp.int32),
        pltpu.VMEM((128,), jnp.float32),
        pltpu.VMEM((RPT,), jnp.float32),
        pltpu.VMEM_SHARED((NP,), jnp.float32),
        pltpu.SemaphoreType.DMA,
    ],
)
def _deg_kernel(dst_hbm, out_hbm, idx_v, ones_v, z_v, acc_sh, sem):
    c = lax.axis_index("c")
    s = lax.axis_index("s")
    wid = s * NC + c
    pltpu.sync_copy(dst_hbm.at[wid], idx_v)
    _fill(ones_v, 128 // L, 1.0, jnp.float32)
    _fill(z_v, RPT // L, 0.0, jnp.float32)
    pltpu.sync_copy(z_v, acc_sh.at[pl.ds(s * RPT, RPT)])
    plsc.subcore_barrier()

    def fire(j, _):
        pltpu.async_copy(ones_v, acc_sh.at[idx_v.at[j]], sem, add=True)
        return 0
    lax.fori_loop(0, NB, fire, 0)

    def drain(j, _):
        pltpu.make_async_copy(ones_v, acc_sh.at[idx_v.at[0]], sem).wait()
        return 0
    lax.fori_loop(0, NB, drain, 0)
    plsc.subcore_barrier()
    pltpu.sync_copy(acc_sh.at[pl.ds(s * RPT, RPT)],
                    out_hbm.at[c, pl.ds(s * RPT, RPT)])


NBP = 27
ZB = 2560


@functools.partial(
    pl.kernel,
    out_type=jax.ShapeDtypeStruct((NC, NP * D), jnp.float32),
    mesh=_mesh,
    scratch_types=[
        pltpu.VMEM((NBP * 128,), jnp.int32),
        pltpu.VMEM((NBP * 128,), jnp.int32),
        pltpu.VMEM((NP,), jnp.float32),
        pltpu.VMEM((NP,), jnp.int32),
        pltpu.VMEM((NBP, 128), jnp.float32),
        pltpu.VMEM((NBP, 128), jnp.int32),
        pltpu.VMEM((NBP, 128), jnp.float32),
        pltpu.VMEM((NBP, 128), jnp.int32),
        pltpu.VMEM((ZB,), jnp.float32),
        pltpu.VMEM_SHARED((NP * D,), jnp.float32),
        pltpu.SemaphoreType.DMA,
        pltpu.SemaphoreType.DMA,
    ],
    compiler_params=pltpu.CompilerParams(needs_layout_passes=False),
)
def _label_kernel(src_hbm, dst_hbm, dis_hbm, x_hbm, out_hbm,
                  src_v, dst_v, dis_v, x_v, val0, fidx0, val1, fidx1,
                  z_v, a_sh, sem0, sem1):
    c = lax.axis_index("c")
    s = lax.axis_index("s")
    wid = s * NC + c
    pltpu.sync_copy(dis_hbm, dis_v)
    pltpu.sync_copy(x_hbm, x_v)

    _fill(z_v, ZB // L, 0.0, jnp.float32)
    sl = RPT * D

    def zcopy(q, _):
        pltpu.sync_copy(z_v, a_sh.at[pl.ds(s * sl + q * ZB, ZB)])
        return 0
    lax.fori_loop(0, sl // ZB, zcopy, 0)
    plsc.subcore_barrier()

    def drain(val_v, fidx_v, sem):
        def dbody(j, _):
            pltpu.make_async_copy(val_v.at[0], a_sh.at[fidx_v.at[0]],
                                  sem).wait()
            return 0
        lax.fori_loop(0, NBP, dbody, 0)

    bufs = ((val0, fidx0, sem0), (val1, fidx1, sem1))
    for p in range(NB // NBP):
        val_v, fidx_v, sem = bufs[p % 2]
        if p >= 2:
            drain(val_v, fidx_v, sem)
        base = wid * EPW + p * (NBP * 128)
        pltpu.sync_copy(src_hbm.at[pl.ds(base, NBP * 128)], src_v)
        pltpu.sync_copy(dst_hbm.at[pl.ds(base, NBP * 128)], dst_v)

        def cbody(i, _):
            row, col = i // 8, (i % 8) * L
            s16 = src_v[pl.ds(i * L, L)]
            d16 = dst_v[pl.ds(i * L, L)]
            dv = plsc.load_gather(dis_v, [s16])
            lb = plsc.load_gather(x_v, [s16])
            val_v[row, pl.ds(col, L)] = dv
            fidx_v[row, pl.ds(col, L)] = d16 * D + lb
            return 0
        lax.fori_loop(0, NBP * 8, cbody, 0)

        def fbody(j, _):
            pltpu.async_copy(val_v.at[j], a_sh.at[fidx_v.at[j]], sem,
                             add=True)
            return 0
        lax.fori_loop(0, NBP, fbody, 0)
    drain(val1, fidx1, sem1)
    drain(val0, fidx0, sem0)
    plsc.subcore_barrier()
    pltpu.sync_copy(a_sh.at[pl.ds(s * sl, sl)],
                    out_hbm.at[c, pl.ds(s * sl, sl)])


NB0 = 64
NB1 = 104
DSEG = 32
SSEG = 56
TOTB = NB0 * NS + NB1 * NS + 24
E3R = TOTB * 128


@functools.partial(
    pl.kernel,
    out_type=jax.ShapeDtypeStruct((NC, NP, D), jnp.float32),
    mesh=_mesh,
    scratch_types=[
        pltpu.VMEM((SSEG, 128), jnp.int32),
        pltpu.VMEM((DSEG, 128), jnp.int32),
        pltpu.VMEM((128, D), jnp.float32),
        pltpu.VMEM((128, D), jnp.float32),
        pltpu.VMEM_SHARED((NP, D), jnp.float32),
        pltpu.SemaphoreType.DMA,
        pltpu.SemaphoreType.DMA,
    ],
    compiler_params=pltpu.CompilerParams(needs_layout_passes=False),
)
def _segsum_kernel(src_hbm, dst_hbm, g1_hbm, out_hbm,
                   src_v, dst_v, rows0, rows1, s_sh, sem0, sem1):
    c = lax.axis_index("c")
    s = lax.axis_index("s")

    def zbody(i, _):
        rows0[i // 8, pl.ds((i % 8) * L, L)] = jnp.zeros((L,), jnp.float32)
        return 0
    lax.fori_loop(0, 128 * 8, zbody, 0)

    def zcopy(q, _):
        pltpu.sync_copy(rows0, s_sh.at[pl.ds(s * RPT + q * 128, 128)])
        return 0
    lax.fori_loop(0, RPT // 128, zcopy, 0)
    plsc.subcore_barrier()

    def run(base, nbc):
        pltpu.sync_copy(src_hbm.at[pl.ds(base, SSEG)], src_v)
        pltpu.sync_copy(dst_hbm.at[pl.ds(base, DSEG)], dst_v)
        pltpu.async_copy(g1_hbm.at[src_v.at[0]], rows0, sem0)

        def pair(jj, _):
            j0 = 2 * jj

            @pl.when((j0 > 0) & (j0 % DSEG == 0))
            def _():
                off = pl.multiple_of(base + j0, 8)
                pltpu.sync_copy(dst_hbm.at[pl.ds(off, DSEG)], dst_v)
            rd = j0 % DSEG
            rs0 = j0 - jnp.where(j0 >= SSEG, SSEG, 0)
            rs1 = j0 + 1 - jnp.where(j0 + 1 >= SSEG, SSEG, 0)
            rs2 = j0 + 2 - jnp.where(j0 + 2 >= SSEG, SSEG, 0)
            pltpu.make_async_copy(g1_hbm.at[src_v.at[rs0]], rows0,
                                  sem0).wait()
            pltpu.async_copy(g1_hbm.at[src_v.at[rs1]], rows1, sem1)
            pltpu.sync_copy(rows0, s_sh.at[dst_v.at[rd]], add=True)
            pltpu.make_async_copy(g1_hbm.at[src_v.at[rs1]], rows1,
                                  sem1).wait()

            @pl.when(j0 + 2 == SSEG)
            def _():
                pltpu.sync_copy(src_hbm.at[pl.ds(base + SSEG, SSEG)],
                                src_v)

            @pl.when(j0 + 2 < nbc)
            def _():
                pltpu.async_copy(g1_hbm.at[src_v.at[rs2]], rows0, sem0)
            pltpu.sync_copy(rows1, s_sh.at[dst_v.at[rd + 1]], add=True)
            return 0
        lax.fori_loop(0, nbc // 2, pair, 0)

    @pl.when(c == 0)
    def _():
        run(s * NB0, NB0)

    @pl.when(c == 1)
    def _():
        run(NB0 * NS + s * NB1, NB1)
    plsc.subcore_barrier()
    pltpu.sync_copy(s_sh.at[pl.ds(s * RPT, RPT)],
                    out_hbm.at[c, pl.ds(s * RPT, RPT)])


def _dis_body(deg_ref, out_ref):
    d = deg_ref[0] + deg_ref[1]
    out_ref[...] = lax.rsqrt(jnp.maximum(d, 1.0))


def _l1_body(a0_ref, a1_ref, dis_ref, w1_ref, b1_ref, out_ref):
    a = a0_ref[...] + a1_ref[...]
    t = jnp.dot(a, w1_ref[...], preferred_element_type=jnp.float32)
    dis = dis_ref[...]
    h = jnp.maximum(dis * t + b1_ref[...], 0.0)
    out_ref[...] = dis * h


def _l2_body(s0_ref, s1_ref, dis_ref, w2_ref, b2_ref, fcw_ref, fcb_ref,
             out_ref):
    p = dis_ref[...] * (s0_ref[...] + s1_ref[...])
    h = jnp.maximum(
        jnp.dot(p, w2_ref[...], preferred_element_type=jnp.float32)
        + b2_ref[...], 0.0)
    out_ref[...] = (
        jnp.dot(h, fcw_ref[...], preferred_element_type=jnp.float32)
        + fcb_ref[...])


RB = 256


def kernel(x, edge_index, W1, b1, W2, b2, fc_W, fc_b):
    x = x.astype(jnp.int32)
    ei = edge_index.astype(jnp.int32)
    iota_n = jnp.arange(N, dtype=jnp.int32)
    pad = jnp.full((E_PAD - E_AUG,), NP - 1, jnp.int32)
    src = jnp.concatenate([ei[0], iota_n, pad]).reshape(NW, NB, 128)
    dst = jnp.concatenate([ei[1], iota_n, pad]).reshape(NW, NB, 128)
    xp = jnp.pad(x, (0, NP - N))

    deg2 = _deg_kernel(dst)

    dis2d = pl.pallas_call(
        _dis_body,
        out_shape=jax.ShapeDtypeStruct((NP // 128, 128), jnp.float32),
        in_specs=[pl.BlockSpec((NC, NP // 128, 128), lambda: (0, 0, 0))],
        out_specs=pl.BlockSpec((NP // 128, 128), lambda: (0, 0)),
    )(deg2.reshape(NC, NP // 128, 128))
    dis = dis2d.reshape(NP)

    a2 = _label_kernel(src.reshape(E_PAD), dst.reshape(E_PAD), dis, xp)

    g1 = pl.pallas_call(
        _l1_body,
        grid=(NP // RB,),
        out_shape=jax.ShapeDtypeStruct((NP, D), jnp.float32),
        in_specs=[
            pl.BlockSpec((RB, D), lambda i: (i, 0)),
            pl.BlockSpec((RB, D), lambda i: (i, 0)),
            pl.BlockSpec((RB, 1), lambda i: (i, 0)),
            pl.BlockSpec((D, D), lambda i: (0, 0)),
            pl.BlockSpec((1, D), lambda i: (0, 0)),
        ],
        out_specs=pl.BlockSpec((RB, D), lambda i: (i, 0)),
    )(a2[0].reshape(NP, D), a2[1].reshape(NP, D), dis.reshape(NP, 1),
      W1, b1.reshape(1, D))

    pad3 = jnp.full((E3R - E_AUG,), NP - 1, jnp.int32)
    src3 = jnp.concatenate([ei[0], iota_n, pad3]).reshape(TOTB, 128)
    dst3 = jnp.concatenate([ei[1], iota_n, pad3]).reshape(TOTB, 128)
    s2 = _segsum_kernel(src3, dst3, g1)

    out = pl.pallas_call(
        _l2_body,
        grid=(NP // RB,),
        out_shape=jax.ShapeDtypeStruct((NP, DO), jnp.float32),
        in_specs=[
            pl.BlockSpec((RB, D), lambda i: (i, 0)),
            pl.BlockSpec((RB, D), lambda i: (i, 0)),
            pl.BlockSpec((RB, 1), lambda i: (i, 0)),
            pl.BlockSpec((D, D), lambda i: (0, 0)),
            pl.BlockSpec((1, D), lambda i: (0, 0)),
            pl.BlockSpec((D, DO), lambda i: (0, 0)),
            pl.BlockSpec((1, DO), lambda i: (0, 0)),
        ],
        out_specs=pl.BlockSpec((RB, DO), lambda i: (i, 0)),
    )(s2[0], s2[1], dis.reshape(NP, 1), W2, b2.reshape(1, D),
      fc_W, fc_b.reshape(1, DO))

    return out[:N]


# speedup vs baseline: 2.4312x; 2.4312x over previous
import functools

import jax
import jax.numpy as jnp
from jax import lax
from jax.experimental import pallas as pl
from jax.experimental.pallas import tpu as pltpu
from jax.experimental.pallas import tpu_sc as plsc

N = 10000
NP = 10240
D = 128
DO = 64
NC, NS, L = 2, 16, 16
NW = NC * NS
NB = 81
EPW = NB * 128
E_PAD = EPW * NW
E_AUG = 320000 + N
RPT = NP // NS
AP_STRIDE = 82944
AP = NS * AP_STRIDE

_mesh = plsc.VectorSubcoreMesh(core_axis_name="c", subcore_axis_name="s")


def _fill(ref, n_vregs, value, dtype):
    def body(i, _):
        ref[pl.ds(i * L, L)] = jnp.full((L,), value, dtype)
        return 0
    lax.fori_loop(0, n_vregs, body, 0)


@functools.partial(
    pl.kernel,
    out_type=jax.ShapeDtypeStruct((NP,), jnp.float32),
    mesh=_mesh,
    scratch_types=[
        pltpu.VMEM((NB, 128), jnp.int32),
        pltpu.VMEM((NB, 128), jnp.int32),
        pltpu.VMEM((128,), jnp.float32),
        pltpu.VMEM((RPT,), jnp.float32),
        pltpu.VMEM_SHARED((NP,), jnp.float32),
        pltpu.SemaphoreType.DMA,
    ],
    compiler_params=pltpu.CompilerParams(needs_layout_passes=False),
)
def _deg_kernel(dst_hbm, out_hbm, idx_v, idx2_v, ones_v, z_v, acc_sh,
                sem):
    c = lax.axis_index("c")
    s = lax.axis_index("s")
    pltpu.sync_copy(dst_hbm.at[2 * s], idx_v)
    _fill(ones_v, 128 // L, 1.0, jnp.float32)
    _fill(z_v, RPT // L, 0.0, jnp.float32)
    pltpu.sync_copy(z_v, acc_sh.at[pl.ds(s * RPT, RPT)])
    plsc.subcore_barrier()

    def fire(j, _):
        pltpu.async_copy(ones_v, acc_sh.at[idx_v.at[j]], sem, add=True)
        return 0
    lax.fori_loop(0, NB, fire, 0)
    pltpu.sync_copy(dst_hbm.at[2 * s + 1], idx2_v)

    def fire2(j, _):
        pltpu.async_copy(ones_v, acc_sh.at[idx2_v.at[j]], sem, add=True)
        return 0
    lax.fori_loop(0, NB, fire2, 0)

    def drain(j, _):
        pltpu.make_async_copy(ones_v, acc_sh.at[idx_v.at[0]], sem).wait()
        return 0
    lax.fori_loop(0, 2 * NB, drain, 0)
    plsc.subcore_barrier()

    pltpu.sync_copy(acc_sh.at[pl.ds(s * RPT, RPT)], z_v)

    def nbody(i, _):
        d = jnp.maximum(z_v[pl.ds(i * L, L)], 1.0)
        ib = plsc.bitcast(d, jnp.int32)
        y = plsc.bitcast(1597463007 - lax.shift_right_logical(ib, 1),
                         jnp.float32)
        for _ in range(3):
            y = y * (1.5 - 0.5 * d * y * y)
        z_v[pl.ds(i * L, L)] = y
        return 0
    lax.fori_loop(0, RPT // L, nbody, 0)

    @pl.when(c == 0)
    def _():
        pltpu.sync_copy(z_v, out_hbm.at[pl.ds(s * RPT, RPT)])


NBP = 27
ZB = 2560


@functools.partial(
    pl.kernel,
    out_type=jax.ShapeDtypeStruct((NC, NP * D), jnp.float32),
    mesh=_mesh,
    scratch_types=[
        pltpu.VMEM((NBP * 128,), jnp.int32),
        pltpu.VMEM((NBP * 128,), jnp.int32),
        pltpu.VMEM((NP,), jnp.float32),
        pltpu.VMEM((NP,), jnp.int32),
        pltpu.VMEM((NBP, 128), jnp.float32),
        pltpu.VMEM((NBP, 128), jnp.int32),
        pltpu.VMEM((NBP, 128), jnp.float32),
        pltpu.VMEM((NBP, 128), jnp.int32),
        pltpu.VMEM((ZB,), jnp.float32),
        pltpu.VMEM_SHARED((NP * D,), jnp.float32),
        pltpu.SemaphoreType.DMA,
        pltpu.SemaphoreType.DMA,
    ],
    compiler_params=pltpu.CompilerParams(needs_layout_passes=False),
)
def _label_kernel(src_hbm, dst_hbm, dis_hbm, x_hbm, out_hbm,
                  src_v, dst_v, dis_v, x_v, val0, fidx0, val1, fidx1,
                  z_v, a_sh, sem0, sem1):
    c = lax.axis_index("c")
    s = lax.axis_index("s")
    wid = s * NC + c
    pltpu.sync_copy(dis_hbm, dis_v)
    pltpu.sync_copy(x_hbm, x_v)

    _fill(z_v, ZB // L, 0.0, jnp.float32)
    sl = RPT * D

    def zcopy(q, _):
        pltpu.sync_copy(z_v, a_sh.at[pl.ds(s * sl + q * ZB, ZB)])
        return 0
    lax.fori_loop(0, sl // ZB, zcopy, 0)
    plsc.subcore_barrier()

    def drain(val_v, fidx_v, sem):
        def dbody(j, _):
            pltpu.make_async_copy(val_v.at[0], a_sh.at[fidx_v.at[0]],
                                  sem).wait()
            return 0
        lax.fori_loop(0, NBP, dbody, 0)

    bufs = ((val0, fidx0, sem0), (val1, fidx1, sem1))
    for p in range(NB // NBP):
        val_v, fidx_v, sem = bufs[p % 2]
        if p >= 2:
            drain(val_v, fidx_v, sem)
        base = wid * EPW + p * (NBP * 128)
        pltpu.sync_copy(src_hbm.at[pl.ds(base, NBP * 128)], src_v)
        pltpu.sync_copy(dst_hbm.at[pl.ds(base, NBP * 128)], dst_v)

        def cbody(i, _):
            row, col = i // 8, (i % 8) * L
            s16 = src_v[pl.ds(i * L, L)]
            d16 = dst_v[pl.ds(i * L, L)]
            dv = plsc.load_gather(dis_v, [s16])
            lb = plsc.load_gather(x_v, [s16])
            val_v[row, pl.ds(col, L)] = dv
            fidx_v[row, pl.ds(col, L)] = d16 * D + lb
            return 0
        lax.fori_loop(0, NBP * 8, cbody, 0)

        def fbody(j, _):
            pltpu.async_copy(val_v.at[j], a_sh.at[fidx_v.at[j]], sem,
                             add=True)
            return 0
        lax.fori_loop(0, NBP, fbody, 0)
    drain(val1, fidx1, sem1)
    drain(val0, fidx0, sem0)
    plsc.subcore_barrier()
    pltpu.sync_copy(a_sh.at[pl.ds(s * sl, sl)],
                    out_hbm.at[c, pl.ds(s * sl, sl)])


SEG = 32


@functools.partial(
    pl.kernel,
    out_type=jax.ShapeDtypeStruct((NC, NP, D), jnp.float32),
    mesh=_mesh,
    scratch_types=[
        pltpu.VMEM((NB, 128), jnp.int32),
        pltpu.VMEM((SEG, 128), jnp.int32),
        pltpu.VMEM((128, D), jnp.float32),
        pltpu.VMEM((128, D), jnp.float32),
        pltpu.VMEM_SHARED((NP, D), jnp.float32),
        pltpu.SemaphoreType.DMA,
        pltpu.SemaphoreType.DMA,
    ],
    compiler_params=pltpu.CompilerParams(needs_layout_passes=False),
)
def _segsum_kernel(src_hbm, dst_hbm, g1_hbm, out_hbm,
                   src_v, dst_v, rows0, rows1, s_sh, sem0, sem1):
    c = lax.axis_index("c")
    s = lax.axis_index("s")
    wid = s * NC + c
    pltpu.sync_copy(src_hbm.at[wid], src_v)
    pltpu.sync_copy(dst_hbm.at[wid, pl.ds(0, SEG)], dst_v)

    def zbody(i, _):
        rows0[i // 8, pl.ds((i % 8) * L, L)] = jnp.zeros((L,), jnp.float32)
        return 0
    lax.fori_loop(0, 128 * 8, zbody, 0)

    def zcopy(q, _):
        pltpu.sync_copy(rows0, s_sh.at[pl.ds(s * RPT + q * 128, 128)])
        return 0
    lax.fori_loop(0, RPT // 128, zcopy, 0)
    plsc.subcore_barrier()

    pltpu.async_copy(g1_hbm.at[src_v.at[0]], rows0, sem0)

    def pair(jj, _):
        j0 = 2 * jj

        @pl.when(jj == SEG // 2)
        def _():
            pltpu.sync_copy(dst_hbm.at[wid, pl.ds(SEG, SEG)], dst_v)

        @pl.when(jj == SEG)
        def _():
            pltpu.sync_copy(dst_hbm.at[wid, pl.ds(2 * SEG, NB - 2 * SEG)],
                            dst_v.at[pl.ds(0, NB - 2 * SEG)])
        r0 = j0 % SEG
        pltpu.make_async_copy(g1_hbm.at[src_v.at[j0]], rows0, sem0).wait()
        pltpu.async_copy(g1_hbm.at[src_v.at[j0 + 1]], rows1, sem1)
        pltpu.sync_copy(rows0, s_sh.at[dst_v.at[r0]], add=True)
        pltpu.make_async_copy(g1_hbm.at[src_v.at[j0 + 1]], rows1,
                              sem1).wait()
        pltpu.async_copy(g1_hbm.at[src_v.at[j0 + 2]], rows0, sem0)
        pltpu.sync_copy(rows1, s_sh.at[dst_v.at[r0 + 1]], add=True)
        return 0
    lax.fori_loop(0, (NB - 1) // 2, pair, 0)
    pltpu.make_async_copy(g1_hbm.at[src_v.at[NB - 1]], rows0, sem0).wait()
    pltpu.sync_copy(rows0, s_sh.at[dst_v.at[(NB - 1) % SEG]], add=True)
    plsc.subcore_barrier()
    pltpu.sync_copy(s_sh.at[pl.ds(s * RPT, RPT)],
                    out_hbm.at[c, pl.ds(s * RPT, RPT)])


def _l1_body(a0_ref, a1_ref, dis_ref, w1_ref, b1_ref, out_ref):
    a = a0_ref[...] + a1_ref[...]
    t = jnp.dot(a, w1_ref[...], preferred_element_type=jnp.float32)
    dis = dis_ref[...]
    h = jnp.maximum(dis * t + b1_ref[...], 0.0)
    out_ref[...] = dis * h


def _l2_body(s0_ref, s1_ref, dis_ref, w2_ref, b2_ref, fcw_ref, fcb_ref,
             out_ref):
    p = dis_ref[...] * (s0_ref[...] + s1_ref[...])
    h = jnp.maximum(
        jnp.dot(p, w2_ref[...], preferred_element_type=jnp.float32)
        + b2_ref[...], 0.0)
    out_ref[...] = (
        jnp.dot(h, fcw_ref[...], preferred_element_type=jnp.float32)
        + fcb_ref[...])


RB = 256


def kernel(x, edge_index, W1, b1, W2, b2, fc_W, fc_b):
    x = x.astype(jnp.int32)
    ei = edge_index.astype(jnp.int32)
    iota_n = jnp.arange(N, dtype=jnp.int32)
    pad = jnp.full((E_PAD - E_AUG,), NP - 1, jnp.int32)
    src = jnp.concatenate([ei[0], iota_n, pad]).reshape(NW, NB, 128)
    dst = jnp.concatenate([ei[1], iota_n, pad]).reshape(NW, NB, 128)
    xp = jnp.pad(x, (0, NP - N))

    dis = _deg_kernel(dst)

    a2 = _label_kernel(src.reshape(E_PAD), dst.reshape(E_PAD), dis, xp)

    g1 = pl.pallas_call(
        _l1_body,
        grid=(NP // RB,),
        out_shape=jax.ShapeDtypeStruct((NP, D), jnp.float32),
        in_specs=[
            pl.BlockSpec((RB, D), lambda i: (i, 0)),
            pl.BlockSpec((RB, D), lambda i: (i, 0)),
            pl.BlockSpec((RB, 1), lambda i: (i, 0)),
            pl.BlockSpec((D, D), lambda i: (0, 0)),
            pl.BlockSpec((1, D), lambda i: (0, 0)),
        ],
        out_specs=pl.BlockSpec((RB, D), lambda i: (i, 0)),
    )(a2[0].reshape(NP, D), a2[1].reshape(NP, D), dis.reshape(NP, 1),
      W1, b1.reshape(1, D))

    s2 = _segsum_kernel(src, dst, g1)

    out = pl.pallas_call(
        _l2_body,
        grid=(NP // RB,),
        out_shape=jax.ShapeDtypeStruct((NP, DO), jnp.float32),
        in_specs=[
            pl.BlockSpec((RB, D), lambda i: (i, 0)),
            pl.BlockSpec((RB, D), lambda i: (i, 0)),
            pl.BlockSpec((RB, 1), lambda i: (i, 0)),
            pl.BlockSpec((D, D), lambda i: (0, 0)),
            pl.BlockSpec((1, D), lambda i: (0, 0)),
            pl.BlockSpec((D, DO), lambda i: (0, 0)),
            pl.BlockSpec((1, DO), lambda i: (0, 0)),
        ],
        out_specs=pl.BlockSpec((RB, DO), lambda i: (i, 0)),
    )(s2[0], s2[1], dis.reshape(NP, 1), W2, b2.reshape(1, D),
      fc_W, fc_b.reshape(1, DO))

    return out[:N]


# speedup vs baseline: 2.4422x; 1.0045x over previous
import functools

import jax
import jax.numpy as jnp
from jax import lax
from jax.experimental import pallas as pl
from jax.experimental.pallas import tpu as pltpu
from jax.experimental.pallas import tpu_sc as plsc

N = 10000
NP = 10240
D = 128
DO = 64
NC, NS, L = 2, 16, 16
NW = NC * NS
NB = 81
EPW = NB * 128
E_PAD = EPW * NW
E_AUG = 320000 + N
RPT = NP // NS
AP_STRIDE = 82944
AP = NS * AP_STRIDE

_mesh = plsc.VectorSubcoreMesh(core_axis_name="c", subcore_axis_name="s")


def _fill(ref, n_vregs, value, dtype):
    def body(i, _):
        ref[pl.ds(i * L, L)] = jnp.full((L,), value, dtype)
        return 0
    lax.fori_loop(0, n_vregs, body, 0)


@functools.partial(
    pl.kernel,
    out_type=jax.ShapeDtypeStruct((NC, NP), jnp.float32),
    mesh=_mesh,
    scratch_types=[
        pltpu.VMEM((NB, 128), jnp.int32),
        pltpu.VMEM((128,), jnp.float32),
        pltpu.VMEM((RPT,), jnp.float32),
        pltpu.VMEM_SHARED((NP,), jnp.float32),
        pltpu.SemaphoreType.DMA,
    ],
)
def _deg_kernel(dst_hbm, out_hbm, idx_v, ones_v, z_v, acc_sh, sem):
    c = lax.axis_index("c")
    s = lax.axis_index("s")
    wid = s * NC + c
    pltpu.sync_copy(dst_hbm.at[wid], idx_v)
    _fill(ones_v, 128 // L, 1.0, jnp.float32)
    _fill(z_v, RPT // L, 0.0, jnp.float32)
    pltpu.sync_copy(z_v, acc_sh.at[pl.ds(s * RPT, RPT)])
    plsc.subcore_barrier()

    def fire(j, _):
        pltpu.async_copy(ones_v, acc_sh.at[idx_v.at[j]], sem, add=True)
        return 0
    lax.fori_loop(0, NB, fire, 0)

    def drain(j, _):
        pltpu.make_async_copy(ones_v, acc_sh.at[idx_v.at[0]], sem).wait()
        return 0
    lax.fori_loop(0, NB, drain, 0)
    plsc.subcore_barrier()
    pltpu.sync_copy(acc_sh.at[pl.ds(s * RPT, RPT)],
                    out_hbm.at[c, pl.ds(s * RPT, RPT)])


NBP = 27
ZB = 2560


@functools.partial(
    pl.kernel,
    out_type=jax.ShapeDtypeStruct((NC, NP * D), jnp.float32),
    mesh=_mesh,
    scratch_types=[
        pltpu.VMEM((NBP * 128,), jnp.int32),
        pltpu.VMEM((NBP * 128,), jnp.int32),
        pltpu.VMEM((NP,), jnp.float32),
        pltpu.VMEM((NP,), jnp.int32),
        pltpu.VMEM((NBP, 128), jnp.float32),
        pltpu.VMEM((NBP, 128), jnp.int32),
        pltpu.VMEM((NBP, 128), jnp.float32),
        pltpu.VMEM((NBP, 128), jnp.int32),
        pltpu.VMEM((ZB,), jnp.float32),
        pltpu.VMEM_SHARED((NP * D,), jnp.float32),
        pltpu.SemaphoreType.DMA,
        pltpu.SemaphoreType.DMA,
    ],
    compiler_params=pltpu.CompilerParams(needs_layout_passes=False),
)
def _label_kernel(src_hbm, dst_hbm, dis_hbm, x_hbm, out_hbm,
                  src_v, dst_v, dis_v, x_v, val0, fidx0, val1, fidx1,
                  z_v, a_sh, sem0, sem1):
    c = lax.axis_index("c")
    s = lax.axis_index("s")
    wid = s * NC + c
    pltpu.sync_copy(dis_hbm, dis_v)
    pltpu.sync_copy(x_hbm, x_v)

    _fill(z_v, ZB // L, 0.0, jnp.float32)
    sl = RPT * D

    def zcopy(q, _):
        pltpu.sync_copy(z_v, a_sh.at[pl.ds(s * sl + q * ZB, ZB)])
        return 0
    lax.fori_loop(0, sl // ZB, zcopy, 0)
    plsc.subcore_barrier()

    def drain(val_v, fidx_v, sem):
        def dbody(j, _):
            pltpu.make_async_copy(val_v.at[0], a_sh.at[fidx_v.at[0]],
                                  sem).wait()
            return 0
        lax.fori_loop(0, NBP, dbody, 0)

    bufs = ((val0, fidx0, sem0), (val1, fidx1, sem1))
    for p in range(NB // NBP):
        val_v, fidx_v, sem = bufs[p % 2]
        if p >= 2:
            drain(val_v, fidx_v, sem)
        base = wid * EPW + p * (NBP * 128)
        pltpu.sync_copy(src_hbm.at[pl.ds(base, NBP * 128)], src_v)
        pltpu.sync_copy(dst_hbm.at[pl.ds(base, NBP * 128)], dst_v)

        def cbody(i, _):
            row, col = i // 8, (i % 8) * L
            s16 = src_v[pl.ds(i * L, L)]
            d16 = dst_v[pl.ds(i * L, L)]
            dv = plsc.load_gather(dis_v, [s16])
            lb = plsc.load_gather(x_v, [s16])
            val_v[row, pl.ds(col, L)] = dv
            fidx_v[row, pl.ds(col, L)] = d16 * D + lb
            return 0
        lax.fori_loop(0, NBP * 8, cbody, 0)

        def fbody(j, _):
            pltpu.async_copy(val_v.at[j], a_sh.at[fidx_v.at[j]], sem,
                             add=True)
            return 0
        lax.fori_loop(0, NBP, fbody, 0)
    drain(val1, fidx1, sem1)
    drain(val0, fidx0, sem0)
    plsc.subcore_barrier()
    pltpu.sync_copy(a_sh.at[pl.ds(s * sl, sl)],
                    out_hbm.at[c, pl.ds(s * sl, sl)])


SEG = 32


@functools.partial(
    pl.kernel,
    out_type=jax.ShapeDtypeStruct((NC, NP, D), jnp.float32),
    mesh=_mesh,
    scratch_types=[
        pltpu.VMEM((NB, 128), jnp.int32),
        pltpu.VMEM((SEG, 128), jnp.int32),
        pltpu.VMEM((128, D), jnp.float32),
        pltpu.VMEM((128, D), jnp.float32),
        pltpu.VMEM_SHARED((NP, D), jnp.float32),
        pltpu.SemaphoreType.DMA,
        pltpu.SemaphoreType.DMA,
    ],
    compiler_params=pltpu.CompilerParams(needs_layout_passes=False),
)
def _segsum_kernel(src_hbm, dst_hbm, g1_hbm, out_hbm,
                   src_v, dst_v, rows0, rows1, s_sh, sem0, sem1):
    c = lax.axis_index("c")
    s = lax.axis_index("s")
    wid = s * NC + c
    pltpu.sync_copy(src_hbm.at[wid], src_v)
    pltpu.sync_copy(dst_hbm.at[wid, pl.ds(0, SEG)], dst_v)

    def zbody(i, _):
        rows0[i // 8, pl.ds((i % 8) * L, L)] = jnp.zeros((L,), jnp.float32)
        return 0
    lax.fori_loop(0, 128 * 8, zbody, 0)

    def zcopy(q, _):
        pltpu.sync_copy(rows0, s_sh.at[pl.ds(s * RPT + q * 128, 128)])
        return 0
    lax.fori_loop(0, RPT // 128, zcopy, 0)
    plsc.subcore_barrier()

    pltpu.async_copy(g1_hbm.at[src_v.at[0]], rows0, sem0)

    def pair(jj, _):
        j0 = 2 * jj

        @pl.when(jj == SEG // 2)
        def _():
            pltpu.sync_copy(dst_hbm.at[wid, pl.ds(SEG, SEG)], dst_v)

        @pl.when(jj == SEG)
        def _():
            pltpu.sync_copy(dst_hbm.at[wid, pl.ds(2 * SEG, NB - 2 * SEG)],
                            dst_v.at[pl.ds(0, NB - 2 * SEG)])
        r0 = j0 % SEG
        pltpu.make_async_copy(g1_hbm.at[src_v.at[j0]], rows0, sem0).wait()
        pltpu.async_copy(g1_hbm.at[src_v.at[j0 + 1]], rows1, sem1)
        pltpu.sync_copy(rows0, s_sh.at[dst_v.at[r0]], add=True)
        pltpu.make_async_copy(g1_hbm.at[src_v.at[j0 + 1]], rows1,
                              sem1).wait()
        pltpu.async_copy(g1_hbm.at[src_v.at[j0 + 2]], rows0, sem0)
        pltpu.sync_copy(rows1, s_sh.at[dst_v.at[r0 + 1]], add=True)
        return 0
    lax.fori_loop(0, (NB - 1) // 2, pair, 0)
    pltpu.make_async_copy(g1_hbm.at[src_v.at[NB - 1]], rows0, sem0).wait()
    pltpu.sync_copy(rows0, s_sh.at[dst_v.at[(NB - 1) % SEG]], add=True)
    plsc.subcore_barrier()
    pltpu.sync_copy(s_sh.at[pl.ds(s * RPT, RPT)],
                    out_hbm.at[c, pl.ds(s * RPT, RPT)])


def _dis_body(deg_ref, out_ref):
    d = deg_ref[0] + deg_ref[1]
    out_ref[...] = lax.rsqrt(jnp.maximum(d, 1.0))


def _l1_body(a0_ref, a1_ref, dis_ref, w1_ref, b1_ref, out_ref):
    a = a0_ref[...] + a1_ref[...]
    t = jnp.dot(a, w1_ref[...], preferred_element_type=jnp.float32)
    dis = dis_ref[...]
    h = jnp.maximum(dis * t + b1_ref[...], 0.0)
    out_ref[...] = dis * h


def _l2_body(s0_ref, s1_ref, dis_ref, w2_ref, b2_ref, fcw_ref, fcb_ref,
             out_ref):
    p = dis_ref[...] * (s0_ref[...] + s1_ref[...])
    h = jnp.maximum(
        jnp.dot(p, w2_ref[...], preferred_element_type=jnp.float32)
        + b2_ref[...], 0.0)
    out_ref[...] = (
        jnp.dot(h, fcw_ref[...], preferred_element_type=jnp.float32)
        + fcb_ref[...])


RB = 256


def kernel(x, edge_index, W1, b1, W2, b2, fc_W, fc_b):
    x = x.astype(jnp.int32)
    ei = edge_index.astype(jnp.int32)
    iota_n = jnp.arange(N, dtype=jnp.int32)
    pad = jnp.full((E_PAD - E_AUG,), NP - 1, jnp.int32)
    src = jnp.concatenate([ei[0], iota_n, pad]).reshape(NW, NB, 128)
    dst = jnp.concatenate([ei[1], iota_n, pad]).reshape(NW, NB, 128)
    xp = jnp.pad(x, (0, NP - N))

    deg2 = _deg_kernel(dst)

    dis2d = pl.pallas_call(
        _dis_body,
        out_shape=jax.ShapeDtypeStruct((NP // 128, 128), jnp.float32),
        in_specs=[pl.BlockSpec((NC, NP // 128, 128), lambda: (0, 0, 0))],
        out_specs=pl.BlockSpec((NP // 128, 128), lambda: (0, 0)),
    )(deg2.reshape(NC, NP // 128, 128))
    dis = dis2d.reshape(NP)

    a2 = _label_kernel(src.reshape(E_PAD), dst.reshape(E_PAD), dis, xp)

    g1 = pl.pallas_call(
        _l1_body,
        grid=(NP // RB,),
        out_shape=jax.ShapeDtypeStruct((NP, D), jnp.float32),
        in_specs=[
            pl.BlockSpec((RB, D), lambda i: (i, 0)),
            pl.BlockSpec((RB, D), lambda i: (i, 0)),
            pl.BlockSpec((RB, 1), lambda i: (i, 0)),
            pl.BlockSpec((D, D), lambda i: (0, 0)),
            pl.BlockSpec((1, D), lambda i: (0, 0)),
        ],
        out_specs=pl.BlockSpec((RB, D), lambda i: (i, 0)),
    )(a2[0].reshape(NP, D), a2[1].reshape(NP, D), dis.reshape(NP, 1),
      W1, b1.reshape(1, D))

    s2 = _segsum_kernel(src, dst, g1)

    out = pl.pallas_call(
        _l2_body,
        grid=(NP // RB,),
        out_shape=jax.ShapeDtypeStruct((NP, DO), jnp.float32),
        in_specs=[
            pl.BlockSpec((RB, D), lambda i: (i, 0)),
            pl.BlockSpec((RB, D), lambda i: (i, 0)),
            pl.BlockSpec((RB, 1), lambda i: (i, 0)),
            pl.BlockSpec((D, D), lambda i: (0, 0)),
            pl.BlockSpec((1, D), lambda i: (0, 0)),
            pl.BlockSpec((D, DO), lambda i: (0, 0)),
            pl.BlockSpec((1, DO), lambda i: (0, 0)),
        ],
        out_specs=pl.BlockSpec((RB, DO), lambda i: (i, 0)),
    )(s2[0], s2[1], dis.reshape(NP, 1), W2, b2.reshape(1, D),
      fc_W, fc_b.reshape(1, DO))

    return out[:N]
